# baseline (speedup 1.0000x reference)
import functools

import jax
import jax.numpy as jnp
from jax import lax
from jax.experimental import pallas as pl
from jax.experimental.pallas import tpu as pltpu
from jax.experimental.pallas import tpu_sc as plsc

_L = 16
_RB = 2048


def _lane_splat(vec, lane):
    idx = jnp.full((_L,), lane, dtype=jnp.int32)
    return vec.at[idx].get(mode="promise_in_bounds")


def _merit_prefix_sc(Cost, Pmax):
    NU = Cost.shape[0]
    mesh = plsc.VectorSubcoreMesh(core_axis_name="c", subcore_axis_name="s",
                                  num_cores=1)

    @functools.partial(
        pl.kernel,
        mesh=mesh,
        out_type=jax.ShapeDtypeStruct((NU,), jnp.float32),
        scratch_types=[
            pltpu.VMEM((NU,), jnp.float32),
            pltpu.VMEM((NU,), jnp.float32),
            pltpu.VMEM((2 * _L,), jnp.float32),
        ],
    )
    def run(cost_hbm, pmax_hbm, cb_hbm, cost_v, pmax_v, acc_v):
        s = lax.axis_index("s")
        pltpu.sync_copy(cost_hbm, cost_v)
        pltpu.sync_copy(pmax_hbm, pmax_v)

        iota = lax.broadcasted_iota(jnp.int32, (_L,), 0)
        base = pl.multiple_of(s * 2 * _L, 2 * _L)
        for gv in range(2):
            g0 = base + gv * _L
            cost_g = cost_v[pl.ds(g0, _L)]
            g_ids = g0 + iota

            def jbody(jg, acc):
                cj = cost_v[pl.ds(jg * _L, _L)]
                pj = pmax_v[pl.ds(jg * _L, _L)]
                jb = jg * _L
                for l in range(_L):
                    cjb = _lane_splat(cj, l)
                    pjb = _lane_splat(pj, l)
                    before = (cjb < cost_g) | (
                        (cjb == cost_g) & (jb + l < g_ids))
                    acc = acc + jnp.where(before, pjb, 0.0)
                return acc

            acc_v[pl.ds(gv * _L, _L)] = lax.fori_loop(
                0, NU // _L, jbody, jnp.zeros((_L,), jnp.float32))
        pltpu.sync_copy(acc_v, cb_hbm.at[pl.ds(base, 2 * _L)])

    return run(Cost, Pmax)


def _dense_body(x_ref, pd_ref, cb_ref, pmax_row_ref, w_ref, out_ref):
    total_d = jnp.sum(pd_ref[...]) - w_ref[0, 0] * x_ref[...]
    out_ref[...] = jnp.clip(total_d - cb_ref[...], 0.0, pmax_row_ref[...])


def kernel(x, Cost, Pd, w_capacity, Pmax):
    B = x.shape[0]
    n_unit = Cost.shape[0]

    cb = _merit_prefix_sc(Cost, Pmax)

    x_col = x.reshape(B, 1)
    pd2d = Pd.reshape(-1, 128)
    cb_row = cb.reshape(1, n_unit)
    pmax_row = Pmax.reshape(1, n_unit)
    w2d = w_capacity.reshape(1, 1)

    grid = (B // _RB,)
    return pl.pallas_call(
        _dense_body,
        grid=grid,
        in_specs=[
            pl.BlockSpec((_RB, 1), lambda i: (i, 0)),
            pl.BlockSpec(pd2d.shape, lambda i: (0, 0)),
            pl.BlockSpec((1, n_unit), lambda i: (0, 0)),
            pl.BlockSpec((1, n_unit), lambda i: (0, 0)),
            pl.BlockSpec((1, 1), lambda i: (0, 0)),
        ],
        out_specs=pl.BlockSpec((_RB, n_unit), lambda i: (i, 0)),
        out_shape=jax.ShapeDtypeStruct((B, n_unit), jnp.float32),
        compiler_params=pltpu.CompilerParams(
            dimension_semantics=("arbitrary",)),
    )(x_col, pd2d, cb_row, pmax_row, w2d)


# speedup vs baseline: 1.0270x; 1.0270x over previous
import functools

import jax
import jax.numpy as jnp
from jax import lax
from jax.experimental import pallas as pl
from jax.experimental.pallas import tpu as pltpu
from jax.experimental.pallas import tpu_sc as plsc

_L = 16
_RB = 4096


def _lane_splat(vec, lane):
    idx = jnp.full((_L,), lane, dtype=jnp.int32)
    return vec.at[idx].get(mode="promise_in_bounds")


def _merit_prefix_sc(Cost, Pmax):
    NU = Cost.shape[0]
    mesh = plsc.VectorSubcoreMesh(core_axis_name="c", subcore_axis_name="s",
                                  num_cores=1)

    @functools.partial(
        pl.kernel,
        mesh=mesh,
        out_type=jax.ShapeDtypeStruct((NU,), jnp.float32),
        scratch_types=[
            pltpu.VMEM((NU,), jnp.float32),
            pltpu.VMEM((NU,), jnp.float32),
            pltpu.VMEM((2 * _L,), jnp.float32),
        ],
    )
    def run(cost_hbm, pmax_hbm, cb_hbm, cost_v, pmax_v, acc_v):
        s = lax.axis_index("s")
        pltpu.sync_copy(cost_hbm, cost_v)
        pltpu.sync_copy(pmax_hbm, pmax_v)

        iota = lax.broadcasted_iota(jnp.int32, (_L,), 0)
        base = pl.multiple_of(s * 2 * _L, 2 * _L)
        for gv in range(2):
            g0 = base + gv * _L
            cost_g = cost_v[pl.ds(g0, _L)]
            g_ids = g0 + iota

            def jbody(jg, acc):
                cj = cost_v[pl.ds(jg * _L, _L)]
                pj = pmax_v[pl.ds(jg * _L, _L)]
                jb = jg * _L
                for l in range(_L):
                    cjb = _lane_splat(cj, l)
                    pjb = _lane_splat(pj, l)
                    before = (cjb < cost_g) | (
                        (cjb == cost_g) & (jb + l < g_ids))
                    acc = acc + jnp.where(before, pjb, 0.0)
                return acc

            acc_v[pl.ds(gv * _L, _L)] = lax.fori_loop(
                0, NU // _L, jbody, jnp.zeros((_L,), jnp.float32))
        pltpu.sync_copy(acc_v, cb_hbm.at[pl.ds(base, 2 * _L)])

    return run(Cost, Pmax)


def _dense_body(x_ref, pd_ref, cb_ref, pmax_row_ref, w_ref, out_ref):
    total_d = jnp.sum(pd_ref[...]) - w_ref[0, 0] * x_ref[...]
    out_ref[...] = jnp.clip(total_d - cb_ref[...], 0.0, pmax_row_ref[...])


def kernel(x, Cost, Pd, w_capacity, Pmax):
    B = x.shape[0]
    n_unit = Cost.shape[0]

    cb = _merit_prefix_sc(Cost, Pmax)

    x_col = x.reshape(B, 1)
    pd2d = Pd.reshape(-1, 128)
    cb_row = cb.reshape(1, n_unit)
    pmax_row = Pmax.reshape(1, n_unit)
    w2d = w_capacity.reshape(1, 1)

    grid = (B // _RB,)
    return pl.pallas_call(
        _dense_body,
        grid=grid,
        in_specs=[
            pl.BlockSpec((_RB, 1), lambda i: (i, 0)),
            pl.BlockSpec(pd2d.shape, lambda i: (0, 0)),
            pl.BlockSpec((1, n_unit), lambda i: (0, 0)),
            pl.BlockSpec((1, n_unit), lambda i: (0, 0)),
            pl.BlockSpec((1, 1), lambda i: (0, 0)),
        ],
        out_specs=pl.BlockSpec((_RB, n_unit), lambda i: (i, 0)),
        out_shape=jax.ShapeDtypeStruct((B, n_unit), jnp.float32),
        compiler_params=pltpu.CompilerParams(
            dimension_semantics=("arbitrary",)),
    )(x_col, pd2d, cb_row, pmax_row, w2d)
